# baseline (speedup 1.0000x reference)
import functools

import jax
import jax.numpy as jnp
from jax import lax
from jax.experimental import pallas as pl
from jax.experimental.pallas import tpu as pltpu
from jax.experimental.pallas import tpu_sc as plsc

B, H, W = 16, 512, 512
NBOX = 20
NWORK = 32
ROWS_PER_W = H // 2
G_ROWS = 16
N_GROUPS = ROWS_PER_W // G_ROWS
WCH = W // 16
NPIX = B * H * W
FG_EXTRA = 12.0


def _floor_i(x):
    t = x.astype(jnp.int32)
    return t - jnp.where(t.astype(jnp.float32) > x, 1, 0)


def _ceil_i(x):
    t = x.astype(jnp.int32)
    return t + jnp.where(x > t.astype(jnp.float32), 1, 0)


def _fg_body(loss_hbm, gt_hbm, out_hbm, bxr_v, buf_v, cov_v, acc_v, sem):
    c = lax.axis_index("c")
    s = lax.axis_index("s")
    wid = s * 2 + c
    batch = wid // 2
    row0 = (wid % 2) * ROWS_PER_W

    pltpu.sync_copy(gt_hbm.at[batch], bxr_v)

    iota = lax.iota(jnp.int32, 16)
    zero_i = jnp.zeros((16,), jnp.int32)
    idxb = jnp.where(iota < NBOX - 16, iota + 16, 0)
    validb = iota < NBOX - 16

    def _col(col, idx):
        return plsc.load_gather(bxr_v, [idx, jnp.full((16,), col, jnp.int32)])

    u1a = _floor_i(_col(0, iota))
    v1a = _floor_i(_col(1, iota))
    u2a = _ceil_i(_col(2, iota))
    v2a = _ceil_i(_col(3, iota))
    u1b = jnp.where(validb, _floor_i(_col(0, idxb)), zero_i)
    v1b = jnp.where(validb, _floor_i(_col(1, idxb)), zero_i)
    u2b = jnp.where(validb, _ceil_i(_col(2, idxb)), zero_i)
    v2b = jnp.where(validb, _ceil_i(_col(3, idxb)), zero_i)

    zero16 = jnp.zeros((16,), jnp.float32)
    acc_v[pl.ds(0, 16)] = zero16

    def group(g, _):
        gmin = row0 + g * G_ROWS
        gmax16 = jnp.full((16,), gmin + G_ROWS - 1, jnp.int32)
        gmin16 = jnp.full((16,), gmin, jnp.int32)
        ghit = jnp.any((v1a <= gmax16) & (v2a > gmin16)) | jnp.any(
            (v1b <= gmax16) & (v2b > gmin16))

        @pl.when(ghit)
        def _group():
            pltpu.make_async_copy(
                loss_hbm.at[batch, pl.ds(gmin, G_ROWS), :], buf_v, sem,
            ).start()
            pltpu.make_async_copy(
                loss_hbm.at[batch, pl.ds(gmin, G_ROWS), :], buf_v, sem,
            ).wait()

            def row_fg(r, _):
                v = gmin + r
                vv = jnp.full((16,), v, jnp.int32)
                acta = (vv >= v1a) & (vv < v2a)
                actb = (vv >= v1b) & (vv < v2b)
                have = jnp.any(acta) | jnp.any(actb)

                @pl.when(have)
                def _fg_row():
                    event = (v == row0) | jnp.any(
                        (vv == v1a) | (vv == v2a) | (vv == v1b) | (vv == v2b))

                    @pl.when(event)
                    def _recompute_cov():
                        eu1a = jnp.where(acta, u1a, 0)
                        eu1b = jnp.where(actb, u1b, 0)
                        eu2a = jnp.where(acta, u2a, 0)
                        eu2b = jnp.where(actb, u2b, 0)
                        bounds = [(eu1a[j], eu2a[j]) for j in range(16)]
                        bounds += [(eu1b[j], eu2b[j])
                                   for j in range(NBOX - 16)]

                        def col_body(ci, _):
                            cols = iota + ci * 16
                            cov = jnp.zeros((16,), jnp.bool_)
                            for e1, e2 in bounds:
                                cov = cov | ((cols >= e1) & (cols < e2))
                            cov_v[pl.ds(ci * 16, 16)] = jnp.where(
                                cov, jnp.float32(FG_EXTRA), jnp.float32(0.0))
                            return 0

                        lax.fori_loop(0, WCH, col_body, 0)

                    vals = [buf_v[r, pl.ds(i * 16, 16)]
                            * cov_v[pl.ds(i * 16, 16)] for i in range(WCH)]
                    while len(vals) > 1:
                        nxt = [vals[i] + vals[i + 1]
                               for i in range(0, len(vals) - 1, 2)]
                        if len(vals) % 2:
                            nxt.append(vals[-1])
                        vals = nxt
                    acc_v[pl.ds(0, 16)] = acc_v[pl.ds(0, 16)] + vals[0]

                return 0

            lax.fori_loop(0, G_ROWS, row_fg, 0)

        return 0

    lax.fori_loop(0, N_GROUPS, group, 0)
    pltpu.sync_copy(acc_v.at[pl.ds(0, 16)], out_hbm.at[pl.ds(wid * 16, 16)])


@functools.cache
def _build_fg_kernel():
    mesh = plsc.VectorSubcoreMesh(core_axis_name="c", subcore_axis_name="s")
    return pl.kernel(
        _fg_body,
        out_type=jax.ShapeDtypeStruct((NWORK * 16,), jnp.float32),
        mesh=mesh,
        compiler_params=pltpu.CompilerParams(
            needs_layout_passes=False, use_tc_tiling_on_sc=True),
        scratch_types=[
            pltpu.VMEM((NBOX, 4), jnp.float32),
            pltpu.VMEM((G_ROWS, W), jnp.float32),
            pltpu.VMEM((W,), jnp.float32),
            pltpu.VMEM((16,), jnp.float32),
            pltpu.SemaphoreType.DMA,
        ],
    )


def _dense_block_sum(x_ref, o_ref):
    i = pl.program_id(0)

    @pl.when(i == 0)
    def _init():
        o_ref[...] = jnp.zeros((8, 128), jnp.float32)

    parts = [jnp.zeros((8, 128), jnp.float32) for _ in range(8)]
    t = 0
    for b in range(x_ref.shape[0]):
        x = x_ref[b]
        for r in range(x.shape[0] // 8):
            for c in range(x.shape[1] // 128):
                parts[t % 8] = parts[t % 8] + x[
                    r * 8:(r + 1) * 8, c * 128:(c + 1) * 128]
                t += 1
    while len(parts) > 1:
        parts = [parts[k] + parts[k + 1] for k in range(0, len(parts), 2)]
    o_ref[...] = o_ref[...] + parts[0]


@functools.cache
def _build_dense_kernel():
    return pl.pallas_call(
        _dense_block_sum,
        grid=(B // 2,),
        in_specs=[pl.BlockSpec((2, H, W), lambda i: (i, 0, 0))],
        out_specs=pl.BlockSpec((8, 128), lambda i: (0, 0)),
        out_shape=jax.ShapeDtypeStruct((8, 128), jnp.float32),
    )


def kernel(loss, gt_boxes2d):
    fg = _build_fg_kernel()(loss, gt_boxes2d)
    dense = _build_dense_kernel()(loss)
    return (dense.sum() + fg.sum()) / jnp.float32(NPIX)


# speedup vs baseline: 1.0379x; 1.0379x over previous
import functools

import jax
import jax.numpy as jnp
from jax import lax
from jax.experimental import pallas as pl
from jax.experimental.pallas import tpu as pltpu
from jax.experimental.pallas import tpu_sc as plsc

B, H, W = 16, 512, 512
NBOX = 20
NWORK = 32
ROWS_PER_W = H // 2
G_ROWS = 16
N_GROUPS = ROWS_PER_W // G_ROWS
WCH = W // 16
NPIX = B * H * W
FG_EXTRA = 12.0


def _floor_i(x):
    t = x.astype(jnp.int32)
    return t - jnp.where(t.astype(jnp.float32) > x, 1, 0)


def _ceil_i(x):
    t = x.astype(jnp.int32)
    return t + jnp.where(x > t.astype(jnp.float32), 1, 0)


def _fg_body(loss_hbm, gt_hbm, out_hbm, bxr_v, buf_v, cov_v, acc_v, sem):
    c = lax.axis_index("c")
    s = lax.axis_index("s")
    wid = s * 2 + c
    batch = wid // 2
    row0 = (wid % 2) * ROWS_PER_W

    pltpu.sync_copy(gt_hbm.at[batch], bxr_v)

    iota = lax.iota(jnp.int32, 16)
    zero_i = jnp.zeros((16,), jnp.int32)
    idxb = jnp.where(iota < NBOX - 16, iota + 16, 0)
    validb = iota < NBOX - 16

    def _col(col, idx):
        return plsc.load_gather(bxr_v, [idx, jnp.full((16,), col, jnp.int32)])

    u1a = _floor_i(_col(0, iota))
    v1a = _floor_i(_col(1, iota))
    u2a = _ceil_i(_col(2, iota))
    v2a = _ceil_i(_col(3, iota))
    u1b = jnp.where(validb, _floor_i(_col(0, idxb)), zero_i)
    v1b = jnp.where(validb, _floor_i(_col(1, idxb)), zero_i)
    u2b = jnp.where(validb, _ceil_i(_col(2, idxb)), zero_i)
    v2b = jnp.where(validb, _ceil_i(_col(3, idxb)), zero_i)

    zero16 = jnp.zeros((16,), jnp.float32)
    acc_v[pl.ds(0, 16)] = zero16

    def group(g, _):
        gmin = row0 + g * G_ROWS
        gmax16 = jnp.full((16,), gmin + G_ROWS - 1, jnp.int32)
        gmin16 = jnp.full((16,), gmin, jnp.int32)
        ghit = jnp.any((v1a <= gmax16) & (v2a > gmin16)) | jnp.any(
            (v1b <= gmax16) & (v2b > gmin16))

        @pl.when(ghit)
        def _group():
            pltpu.make_async_copy(
                loss_hbm.at[batch, pl.ds(gmin, G_ROWS), :], buf_v, sem,
            ).start()
            pltpu.make_async_copy(
                loss_hbm.at[batch, pl.ds(gmin, G_ROWS), :], buf_v, sem,
            ).wait()

            def row_fg(r, _):
                v = gmin + r
                vv = jnp.full((16,), v, jnp.int32)
                acta = (vv >= v1a) & (vv < v2a)
                actb = (vv >= v1b) & (vv < v2b)
                have = jnp.any(acta) | jnp.any(actb)

                @pl.when(have)
                def _fg_row():
                    event = (v == row0) | jnp.any(
                        (vv == v1a) | (vv == v2a) | (vv == v1b) | (vv == v2b))

                    @pl.when(event)
                    def _recompute_cov():
                        eu1a = jnp.where(acta, u1a, 0)
                        eu1b = jnp.where(actb, u1b, 0)
                        eu2a = jnp.where(acta, u2a, 0)
                        eu2b = jnp.where(actb, u2b, 0)
                        bounds = [(eu1a[j], eu2a[j]) for j in range(16)]
                        bounds += [(eu1b[j], eu2b[j])
                                   for j in range(NBOX - 16)]

                        def col_body(ci, _):
                            cols = iota + ci * 16
                            cov = jnp.zeros((16,), jnp.bool_)
                            for e1, e2 in bounds:
                                cov = cov | ((cols >= e1) & (cols < e2))
                            cov_v[pl.ds(ci * 16, 16)] = jnp.where(
                                cov, jnp.float32(FG_EXTRA), jnp.float32(0.0))
                            return 0

                        lax.fori_loop(0, WCH, col_body, 0)

                    vals = [buf_v[r, pl.ds(i * 16, 16)]
                            * cov_v[pl.ds(i * 16, 16)] for i in range(WCH)]
                    while len(vals) > 1:
                        nxt = [vals[i] + vals[i + 1]
                               for i in range(0, len(vals) - 1, 2)]
                        if len(vals) % 2:
                            nxt.append(vals[-1])
                        vals = nxt
                    acc_v[pl.ds(0, 16)] = acc_v[pl.ds(0, 16)] + vals[0]

                return 0

            lax.fori_loop(0, G_ROWS, row_fg, 0)

        return 0

    lax.fori_loop(0, N_GROUPS, group, 0)
    pltpu.sync_copy(acc_v.at[pl.ds(0, 16)], out_hbm.at[pl.ds(wid * 16, 16)])


@functools.cache
def _build_fg_kernel():
    mesh = plsc.VectorSubcoreMesh(core_axis_name="c", subcore_axis_name="s")
    return pl.kernel(
        _fg_body,
        out_type=jax.ShapeDtypeStruct((NWORK * 16,), jnp.float32),
        mesh=mesh,
        compiler_params=pltpu.CompilerParams(
            needs_layout_passes=False, use_tc_tiling_on_sc=True),
        scratch_types=[
            pltpu.VMEM((NBOX, 4), jnp.float32),
            pltpu.VMEM((G_ROWS, W), jnp.float32),
            pltpu.VMEM((W,), jnp.float32),
            pltpu.VMEM((16,), jnp.float32),
            pltpu.SemaphoreType.DMA,
        ],
    )


def _dense_block_sum(x_ref, o_ref):
    i = pl.program_id(0)

    @pl.when(i == 0)
    def _init():
        o_ref[...] = jnp.zeros((8, 128), jnp.float32)

    parts = [jnp.zeros((8, 128), jnp.float32) for _ in range(8)]
    t = 0
    for b in range(x_ref.shape[0]):
        x = x_ref[b]
        for r in range(x.shape[0] // 8):
            for c in range(x.shape[1] // 128):
                parts[t % 8] = parts[t % 8] + x[
                    r * 8:(r + 1) * 8, c * 128:(c + 1) * 128]
                t += 1
    while len(parts) > 1:
        parts = [parts[k] + parts[k + 1] for k in range(0, len(parts), 2)]
    o_ref[...] = o_ref[...] + parts[0]


@functools.cache
def _build_dense_kernel():
    return pl.pallas_call(
        _dense_block_sum,
        grid=(B // 4,),
        in_specs=[pl.BlockSpec((4, H, W), lambda i: (i, 0, 0))],
        out_specs=pl.BlockSpec((8, 128), lambda i: (0, 0)),
        out_shape=jax.ShapeDtypeStruct((8, 128), jnp.float32),
    )


def kernel(loss, gt_boxes2d):
    fg = _build_fg_kernel()(loss, gt_boxes2d)
    dense = _build_dense_kernel()(loss)
    return (dense.sum() + fg.sum()) / jnp.float32(NPIX)
